# baseline (speedup 1.0000x reference)
import jax
import jax.numpy as jnp
from jax import lax
from jax.experimental import pallas as pl
from jax.experimental.pallas import tpu as pltpu
from jax.experimental.pallas import tpu_sc as plsc

_B, _C, _H, _W, _N = 4, 768, 224, 224, 4096
_NW = 32
_L = 16
_PTS = _N // 8
_KC = _C // 128
_CP = 32
_NCH = _PTS // _CP
_RC = _CP * _KC
_V = _B * _H * (_W // 8) * _KC * 8


def _sc_body(o1_hbm, o2_hbm, m1_hbm, m2_hbm, out_hbm,
             m1_v, m2_v, base_v, idx1_v, idx2_v,
             v1a, v1b, v2a, v2b, acc_v,
             sem1, sem2):
    cid = lax.axis_index("c")
    sid = lax.axis_index("s")
    wid = sid * 2 + cid
    b = wid // 8
    n0 = (wid % 8) * _PTS

    pltpu.sync_copy(m1_hbm.at[b, pl.ds(n0, _PTS)], m1_v)
    pltpu.sync_copy(m2_hbm.at[b, pl.ds(n0, _PTS)], m2_v)

    bh = b * _H

    def build_idx(m_v, idx_v):
        def base_body(t, u):
            s = t * _L
            m = m_v[pl.ds(s, _L)]
            r = m >> 16
            c = m & 0xFFFF
            base_v[pl.ds(s, _L)] = ((bh + r) * (_W // 8) + (c >> 3)) \
                * (_KC * 8) + (c & 7)
            return u

        lax.fori_loop(0, _PTS // _L, base_body, 0, unroll=4)
        for k in range(_KC):
            def k_body(t, u, k=k):
                s = t * _L
                ci = t >> 1
                off = ci * _RC + k * _CP + (t & 1) * _L
                idx_v[pl.ds(off, _L)] = base_v[pl.ds(s, _L)] + (k * 8)
                return u

            lax.fori_loop(0, _PTS // _L, k_body, 0, unroll=4)

    build_idx(m1_v, idx1_v)
    build_idx(m2_v, idx2_v)

    acc_v[...] = jnp.zeros((_L,), jnp.float32)

    def issue(ci, v1_ref, v2_ref):
        s = ci * _RC
        pltpu.async_copy(o1_hbm.at[idx1_v.at[pl.ds(s, _RC)]], v1_ref, sem1)
        pltpu.async_copy(o2_hbm.at[idx2_v.at[pl.ds(s, _RC)]], v2_ref, sem2)

    def drain_acc(ci, v1_ref, v2_ref):
        s = ci * _RC
        pltpu.make_async_copy(
            o1_hbm.at[idx1_v.at[pl.ds(s, _RC)]], v1_ref, sem1).wait()
        pltpu.make_async_copy(
            o2_hbm.at[idx2_v.at[pl.ds(s, _RC)]], v2_ref, sem2).wait()

        def body(t, a):
            j = t >> 3
            w = (t & 7) << 4
            d = v1_ref[j, pl.ds(w, _L)] - v2_ref[j, pl.ds(w, _L)]
            return a + d * d

        acc = lax.fori_loop(0, _RC * 8, body,
                            jnp.zeros((_L,), jnp.float32), unroll=8)
        acc_v[...] = acc_v[...] + acc

    issue(0, v1a, v2a)

    def pair_body(i, u):
        j = 2 * i
        issue(j + 1, v1b, v2b)
        drain_acc(j, v1a, v2a)
        issue(j + 2, v1a, v2a)
        drain_acc(j + 1, v1b, v2b)
        return u

    lax.fori_loop(0, (_NCH - 2) // 2, pair_body, 0)
    issue(_NCH - 1, v1b, v2b)
    drain_acc(_NCH - 2, v1a, v2a)
    drain_acc(_NCH - 1, v1b, v2b)

    pltpu.sync_copy(acc_v, out_hbm.at[wid])


@jax.jit
def _sc_loss(o1, o2, m1, m2):
    mesh = plsc.VectorSubcoreMesh(core_axis_name="c", subcore_axis_name="s")
    parts = pl.kernel(
        _sc_body,
        out_type=jax.ShapeDtypeStruct((_NW, _L), jnp.float32),
        mesh=mesh,
        compiler_params=pltpu.CompilerParams(needs_layout_passes=False),
        scratch_types=[
            pltpu.VMEM((_PTS,), jnp.int32),
            pltpu.VMEM((_PTS,), jnp.int32),
            pltpu.VMEM((_PTS,), jnp.int32),
            pltpu.VMEM((_PTS * _KC,), jnp.int32),
            pltpu.VMEM((_PTS * _KC,), jnp.int32),
            pltpu.VMEM((_RC, 128), jnp.float32),
            pltpu.VMEM((_RC, 128), jnp.float32),
            pltpu.VMEM((_RC, 128), jnp.float32),
            pltpu.VMEM((_RC, 128), jnp.float32),
            pltpu.VMEM((_L,), jnp.float32),
            pltpu.SemaphoreType.DMA,
            pltpu.SemaphoreType.DMA,
        ],
    )(o1, o2, m1, m2)
    return jnp.sum(parts) * (1.0 / (_B * _N))


def kernel(out_1, out_2, match_1, match_2, nonmatch_2):
    del nonmatch_2

    def rowview(x):
        y = x.transpose(0, 2, 3, 1)
        y = y.reshape(_B, _H, _W // 8, 8, _KC, 128)
        y = y.transpose(0, 1, 2, 4, 3, 5)
        return y.reshape(_V, 128)

    m1 = (match_1[:, :, 0] << 16) | match_1[:, :, 1]
    m2 = (match_2[:, :, 0] << 16) | match_2[:, :, 1]
    return _sc_loss(rowview(out_1), rowview(out_2), m1, m2)
